# baseline (speedup 1.0000x reference)
import math

import jax
import jax.numpy as jnp
from jax import lax
from jax.experimental import pallas as pl
from jax.experimental.pallas import tpu as pltpu
from jax.experimental.pallas import tpu_sc as plsc

N = 10000
E = 320000
IN_DIM = 128
H = 8
D = 16
EPS = 1e-09
SCALE = 1.0 / math.sqrt(D)

NC = 2
NS = 16
NW = NC * NS
EPW = E // NW
CH = 40
NCHUNK = EPW // CH
IB = 8
NCHUNK_PAD = 256
NBLK = NCHUNK_PAD // IB
WB = 40
NWB = N // WB
WB_PER_TILE = -(-NWB // NS)
CW = H * D + D


def _proj_body(x_ref, wq_ref, bq_ref, wk_ref, bk_ref, wv_ref, bv_ref,
               q_ref, kv_ref):
    xb = x_ref[...]
    q_ref[...] = jnp.dot(xb, wq_ref[...],
                         preferred_element_type=jnp.float32) + bq_ref[...]
    kv_ref[:, :H * D] = jnp.dot(xb, wk_ref[...],
                                preferred_element_type=jnp.float32) + bk_ref[...]
    kv_ref[:, H * D:] = jnp.dot(xb, wv_ref[...],
                                preferred_element_type=jnp.float32) + bv_ref[...]


def _node_proj(x, WQ, bQ, WK, bK, WV, bV):
    blk = 1000
    grid = N // blk
    wspec = pl.BlockSpec((IN_DIM, H * D), lambda i: (0, 0))
    bspec = pl.BlockSpec((1, H * D), lambda i: (0, 0))
    return pl.pallas_call(
        _proj_body,
        grid=(grid,),
        in_specs=[pl.BlockSpec((blk, IN_DIM), lambda i: (i, 0)),
                  wspec, bspec, wspec, bspec, wspec, bspec],
        out_specs=[pl.BlockSpec((blk, H * D), lambda i: (i, 0)),
                   pl.BlockSpec((blk, 2 * H * D), lambda i: (i, 0))],
        out_shape=[jax.ShapeDtypeStruct((N, H * D), jnp.float32),
                   jax.ShapeDtypeStruct((N, 2 * H * D), jnp.float32)],
    )(x, WQ, bQ.reshape(1, -1), WK, bK.reshape(1, -1), WV, bV.reshape(1, -1))


def _edge_proj_body(ea_ref, we_ref, be_ref, eh_ref):
    eh_ref[...] = jnp.dot(ea_ref[...], we_ref[...],
                          preferred_element_type=jnp.float32) + be_ref[...]


def _edge_proj(edge_attr, WE, bE):
    blk = 4000
    grid = E // blk
    return pl.pallas_call(
        _edge_proj_body,
        grid=(grid,),
        in_specs=[pl.BlockSpec((blk, IN_DIM), lambda i: (i, 0)),
                  pl.BlockSpec((IN_DIM, H * D), lambda i: (0, 0)),
                  pl.BlockSpec((1, H * D), lambda i: (0, 0))],
        out_specs=pl.BlockSpec((blk, H * D), lambda i: (i, 0)),
        out_shape=jax.ShapeDtypeStruct((E, H * D), jnp.float32),
    )(edge_attr, WE, bE.reshape(1, -1))


def _edge_kernel4(src2_hbm, dst2_hbm, qh_hbm, kvh_hbm, eh_hbm,
                  oacc_hbm,
                  si_v, di_v, kv_v, q_v, e_v, c_v,
                  semg0, semg1, semq,
                  acc_sh):
    cid = lax.axis_index("c")
    sid = lax.axis_index("s")
    wid = sid * NC + cid
    semg = (semg0, semg1)

    zero16 = jnp.zeros((16,), jnp.float32)

    def _zrow(r, carry):
        for cc in range(CW // 16):
            c_v[r, pl.ds(cc * 16, 16)] = zero16
        return carry
    lax.fori_loop(0, WB, _zrow, 0)

    def _initj(j, carry):
        ci = sid + j * NS
        @pl.when(ci < NWB)
        def _init():
            r0 = pl.multiple_of(ci * WB, 8)
            pltpu.sync_copy(c_v, acc_sh.at[pl.ds(r0, WB)])
        return carry
    lax.fori_loop(0, WB_PER_TILE, _initj, 0)
    plsc.subcore_barrier()

    lane = lax.iota(jnp.int32, 16)
    onehot = [(lane == h).astype(jnp.float32) for h in range(H)]

    ebase = wid * EPW
    crow0 = wid * NCHUNK_PAD

    def _prefetch(blk, c, b):
        pltpu.async_copy(kvh_hbm.at[si_v.at[c]], kv_v.at[b], semg[b])

    def _wait_gathers(b):
        pltpu.make_async_copy(kvh_hbm.at[si_v.at[0]], kv_v.at[b], semg[b]).wait()

    def _stage(blk, c, b):
        cp_q = pltpu.async_copy(qh_hbm.at[di_v.at[c]], q_v, semq)
        @pl.when((c + 1 < IB) & (blk * IB + c + 1 < NCHUNK))
        def _pf():
            _prefetch(blk, c + 1, 1 - b)
        e0 = ebase + (blk * IB + c) * CH
        pltpu.sync_copy(eh_hbm.at[pl.ds(e0, CH)], e_v)
        _wait_gathers(b)
        cp_q.wait()
        kvb, qb, eb = kv_v.at[b], q_v, e_v

        def _dots(e, ecarry):
            srow = zero16
            for h in range(H):
                sl = pl.ds(h * 16, 16)
                vsl = pl.ds(H * D + h * 16, 16)
                dot = jnp.sum(kvb[e, sl] * qb[e, sl] * eb[e, sl]) * SCALE
                svec = jnp.exp(jnp.broadcast_to(dot, (16,)))
                c_v[e, sl] = kvb[e, vsl] * svec
                srow = srow + svec * onehot[h]
            c_v[e, pl.ds(H * D, 16)] = srow
            return ecarry
        lax.fori_loop(0, CH, _dots, 0)

        pltpu.sync_copy(c_v, acc_sh.at[di_v.at[c]], add=True)

    def _block(blk, carry):
        r0 = pl.multiple_of(crow0 + blk * IB, 8)
        pltpu.sync_copy(src2_hbm.at[pl.ds(r0, IB)], si_v)
        pltpu.sync_copy(dst2_hbm.at[pl.ds(r0, IB)], di_v)
        _prefetch(blk, 0, 0)

        def _pair(i, pcarry):
            c0 = i * 2
            @pl.when(blk * IB + c0 < NCHUNK)
            def _s0():
                _stage(blk, c0, 0)
            @pl.when(blk * IB + c0 + 1 < NCHUNK)
            def _s1():
                _stage(blk, c0 + 1, 1)
            return pcarry
        lax.fori_loop(0, IB // 2, _pair, 0)
        return carry
    lax.fori_loop(0, NBLK, _block, 0)

    plsc.subcore_barrier()

    def _wbj(j, carry):
        ci = sid + j * NS
        @pl.when(ci < NWB)
        def _wb():
            r0 = pl.multiple_of(ci * WB, 8)
            ro = pl.multiple_of(cid * N + r0, 8)
            pltpu.sync_copy(acc_sh.at[pl.ds(r0, WB)], c_v)
            pltpu.sync_copy(c_v, oacc_hbm.at[pl.ds(ro, WB)])
        return carry
    lax.fori_loop(0, WB_PER_TILE, _wbj, 0)


def _edge_stage4(src2, dst2, Qh, KVh, Eh):
    mesh = plsc.VectorSubcoreMesh(core_axis_name="c", subcore_axis_name="s")
    f = pl.kernel(
        _edge_kernel4,
        out_type=[jax.ShapeDtypeStruct((NC * N, CW), jnp.float32)],
        mesh=mesh,
        compiler_params=pltpu.CompilerParams(needs_layout_passes=False,
                                             use_tc_tiling_on_sc=False),
        scratch_types=[
            pltpu.VMEM((IB, CH), jnp.int32),
            pltpu.VMEM((IB, CH), jnp.int32),
            pltpu.VMEM((2, CH, 2 * H * D), jnp.float32),
            pltpu.VMEM((CH, H * D), jnp.float32),
            pltpu.VMEM((CH, H * D), jnp.float32),
            pltpu.VMEM((CH, CW), jnp.float32),
            pltpu.SemaphoreType.DMA,
            pltpu.SemaphoreType.DMA,
            pltpu.SemaphoreType.DMA,
            pltpu.VMEM_SHARED((N, CW), jnp.float32),
        ],
    )
    return f(src2, dst2, Qh, KVh, Eh)


def _final_body(acc_ref, out_ref):
    acc = acc_ref[0] + acc_ref[1]
    for h in range(H):
        denom = acc[:, H * D + h:H * D + h + 1] + EPS
        out_ref[:, h * D:(h + 1) * D] = acc[:, h * D:(h + 1) * D] / denom


def _finalize(oacc):
    blk = 1000
    grid = N // blk
    acc2 = oacc.reshape(NC, N, CW)
    return pl.pallas_call(
        _final_body,
        grid=(grid,),
        in_specs=[pl.BlockSpec((NC, blk, CW), lambda i: (0, i, 0))],
        out_specs=pl.BlockSpec((blk, H * D), lambda i: (i, 0)),
        out_shape=jax.ShapeDtypeStruct((N, H * D), jnp.float32),
    )(acc2)


def kernel(x, edge_attr, edge_index, WQ, bQ, WK, bK, WV, bV, WE, bE):
    Qh, KVh = _node_proj(x, WQ, bQ, WK, bK, WV, bV)
    Eh = _edge_proj(edge_attr, WE, bE)
    src = edge_index[0]
    dst = edge_index[1]
    pad = ((0, 0), (0, NCHUNK_PAD - NCHUNK), (0, 0))
    src3 = jnp.pad(src.reshape(NW, NCHUNK, CH), pad).reshape(NW * NCHUNK_PAD, CH)
    dst3 = jnp.pad(dst.reshape(NW, NCHUNK, CH), pad).reshape(NW * NCHUNK_PAD, CH)
    oacc, = _edge_stage4(src3, dst3, Qh, KVh, Eh)
    out = _finalize(oacc)
    return out.reshape(N, H, D)


# speedup vs baseline: 2.1380x; 2.1380x over previous
import math

import jax
import jax.numpy as jnp
from jax import lax
from jax.experimental import pallas as pl
from jax.experimental.pallas import tpu as pltpu
from jax.experimental.pallas import tpu_sc as plsc

N = 10000
E = 320000
IN_DIM = 128
H = 8
D = 16
EPS = 1e-09
SCALE = 1.0 / math.sqrt(D)

NC = 2
NS = 16
NW = NC * NS
EPW = E // NW
CH = 40
NCHUNK = EPW // CH
IB = 8
NCHUNK_PAD = 256
NBLK = NCHUNK_PAD // IB
WB = 40
NWB = N // WB
WB_PER_TILE = -(-NWB // NS)
CW = H * D + D


def _proj_body(x_ref, wq_ref, bq_ref, wk_ref, bk_ref, wv_ref, bv_ref,
               q_ref, kv_ref):
    xb = x_ref[...]
    q_ref[...] = jnp.dot(xb, wq_ref[...],
                         preferred_element_type=jnp.float32) + bq_ref[...]
    kv_ref[:, :H * D] = jnp.dot(xb, wk_ref[...],
                                preferred_element_type=jnp.float32) + bk_ref[...]
    kv_ref[:, H * D:] = jnp.dot(xb, wv_ref[...],
                                preferred_element_type=jnp.float32) + bv_ref[...]


def _node_proj(x, WQ, bQ, WK, bK, WV, bV):
    blk = 1000
    grid = N // blk
    wspec = pl.BlockSpec((IN_DIM, H * D), lambda i: (0, 0))
    bspec = pl.BlockSpec((1, H * D), lambda i: (0, 0))
    return pl.pallas_call(
        _proj_body,
        grid=(grid,),
        in_specs=[pl.BlockSpec((blk, IN_DIM), lambda i: (i, 0)),
                  wspec, bspec, wspec, bspec, wspec, bspec],
        out_specs=[pl.BlockSpec((blk, H * D), lambda i: (i, 0)),
                   pl.BlockSpec((blk, 2 * H * D), lambda i: (i, 0))],
        out_shape=[jax.ShapeDtypeStruct((N, H * D), jnp.float32),
                   jax.ShapeDtypeStruct((N, 2 * H * D), jnp.float32)],
    )(x, WQ, bQ.reshape(1, -1), WK, bK.reshape(1, -1), WV, bV.reshape(1, -1))


def _edge_proj_body(ea_ref, we_ref, be_ref, eh_ref):
    eh_ref[...] = jnp.dot(ea_ref[...], we_ref[...],
                          preferred_element_type=jnp.float32) + be_ref[...]


def _edge_proj(edge_attr, WE, bE):
    blk = 4000
    grid = E // blk
    return pl.pallas_call(
        _edge_proj_body,
        grid=(grid,),
        in_specs=[pl.BlockSpec((blk, IN_DIM), lambda i: (i, 0)),
                  pl.BlockSpec((IN_DIM, H * D), lambda i: (0, 0)),
                  pl.BlockSpec((1, H * D), lambda i: (0, 0))],
        out_specs=pl.BlockSpec((blk, H * D), lambda i: (i, 0)),
        out_shape=jax.ShapeDtypeStruct((E, H * D), jnp.float32),
    )(edge_attr, WE, bE.reshape(1, -1))


def _edge_kernel4(src2_hbm, dst2_hbm, qh_hbm, kvh_hbm, eh_hbm,
                  oacc_hbm,
                  si_v, di_v, kv_v, q_v, e_v, c_v,
                  semg0, semg1, semq,
                  acc_sh):
    cid = lax.axis_index("c")
    sid = lax.axis_index("s")
    wid = sid * NC + cid
    semg = (semg0, semg1)

    zero16 = jnp.zeros((16,), jnp.float32)

    def _zrow(r, carry):
        for cc in range(CW // 16):
            c_v[r, pl.ds(cc * 16, 16)] = zero16
        return carry
    lax.fori_loop(0, WB, _zrow, 0)

    def _initj(j, carry):
        ci = sid + j * NS
        @pl.when(ci < NWB)
        def _init():
            r0 = pl.multiple_of(ci * WB, 8)
            pltpu.sync_copy(c_v, acc_sh.at[pl.ds(r0, WB)])
        return carry
    lax.fori_loop(0, WB_PER_TILE, _initj, 0)
    plsc.subcore_barrier()

    lane = lax.iota(jnp.int32, 16)
    onehot = [(lane == h).astype(jnp.float32) for h in range(H)]
    perms = [lane ^ st for st in (8, 4, 2, 1)]
    bidx = [lane * 0 + h for h in range(H)]

    ebase = wid * EPW
    crow0 = wid * NCHUNK_PAD

    def _prefetch(blk, c, b):
        pltpu.async_copy(kvh_hbm.at[si_v.at[c]], kv_v.at[b], semg[b])

    def _wait_gathers(b):
        pltpu.make_async_copy(kvh_hbm.at[si_v.at[0]], kv_v.at[b], semg[b]).wait()

    def _stage(blk, c, b):
        cp_q = pltpu.async_copy(qh_hbm.at[di_v.at[c]], q_v, semq)
        @pl.when((c + 1 < IB) & (blk * IB + c + 1 < NCHUNK))
        def _pf():
            _prefetch(blk, c + 1, 1 - b)
        e0 = ebase + (blk * IB + c) * CH
        pltpu.sync_copy(eh_hbm.at[pl.ds(e0, CH)], e_v)
        _wait_gathers(b)
        cp_q.wait()
        kvb, qb, eb = kv_v.at[b], q_v, e_v

        def _dots(e, ecarry):
            p = [kvb[e, pl.ds(h * 16, 16)] * qb[e, pl.ds(h * 16, 16)]
                 * eb[e, pl.ds(h * 16, 16)] for h in range(H)]
            for pm in perms:
                p = [ph + jnp.take(ph, pm) for ph in p]
            srow = p[0] * onehot[0]
            for h in range(1, H):
                srow = srow + p[h] * onehot[h]
            svec_all = jnp.exp(srow * SCALE)
            c_v[e, pl.ds(H * D, 16)] = svec_all
            for h in range(H):
                sv = jnp.take(svec_all, bidx[h])
                c_v[e, pl.ds(h * 16, 16)] = kvb[e, pl.ds(H * D + h * 16, 16)] * sv
            return ecarry
        lax.fori_loop(0, CH, _dots, 0)

        pltpu.sync_copy(c_v, acc_sh.at[di_v.at[c]], add=True)

    def _block(blk, carry):
        r0 = pl.multiple_of(crow0 + blk * IB, 8)
        pltpu.sync_copy(src2_hbm.at[pl.ds(r0, IB)], si_v)
        pltpu.sync_copy(dst2_hbm.at[pl.ds(r0, IB)], di_v)
        _prefetch(blk, 0, 0)

        def _pair(i, pcarry):
            c0 = i * 2
            @pl.when(blk * IB + c0 < NCHUNK)
            def _s0():
                _stage(blk, c0, 0)
            @pl.when(blk * IB + c0 + 1 < NCHUNK)
            def _s1():
                _stage(blk, c0 + 1, 1)
            return pcarry
        lax.fori_loop(0, IB // 2, _pair, 0)
        return carry
    lax.fori_loop(0, NBLK, _block, 0)

    plsc.subcore_barrier()

    def _wbj(j, carry):
        ci = sid + j * NS
        @pl.when(ci < NWB)
        def _wb():
            r0 = pl.multiple_of(ci * WB, 8)
            ro = pl.multiple_of(cid * N + r0, 8)
            pltpu.sync_copy(acc_sh.at[pl.ds(r0, WB)], c_v)
            pltpu.sync_copy(c_v, oacc_hbm.at[pl.ds(ro, WB)])
        return carry
    lax.fori_loop(0, WB_PER_TILE, _wbj, 0)


def _edge_stage4(src2, dst2, Qh, KVh, Eh):
    mesh = plsc.VectorSubcoreMesh(core_axis_name="c", subcore_axis_name="s")
    f = pl.kernel(
        _edge_kernel4,
        out_type=[jax.ShapeDtypeStruct((NC * N, CW), jnp.float32)],
        mesh=mesh,
        compiler_params=pltpu.CompilerParams(needs_layout_passes=False,
                                             use_tc_tiling_on_sc=False),
        scratch_types=[
            pltpu.VMEM((IB, CH), jnp.int32),
            pltpu.VMEM((IB, CH), jnp.int32),
            pltpu.VMEM((2, CH, 2 * H * D), jnp.float32),
            pltpu.VMEM((CH, H * D), jnp.float32),
            pltpu.VMEM((CH, H * D), jnp.float32),
            pltpu.VMEM((CH, CW), jnp.float32),
            pltpu.SemaphoreType.DMA,
            pltpu.SemaphoreType.DMA,
            pltpu.SemaphoreType.DMA,
            pltpu.VMEM_SHARED((N, CW), jnp.float32),
        ],
    )
    return f(src2, dst2, Qh, KVh, Eh)


def _final_body(acc_ref, out_ref):
    acc = acc_ref[0] + acc_ref[1]
    for h in range(H):
        denom = acc[:, H * D + h:H * D + h + 1] + EPS
        out_ref[:, h * D:(h + 1) * D] = acc[:, h * D:(h + 1) * D] / denom


def _finalize(oacc):
    blk = 1000
    grid = N // blk
    acc2 = oacc.reshape(NC, N, CW)
    return pl.pallas_call(
        _final_body,
        grid=(grid,),
        in_specs=[pl.BlockSpec((NC, blk, CW), lambda i: (0, i, 0))],
        out_specs=pl.BlockSpec((blk, H * D), lambda i: (i, 0)),
        out_shape=jax.ShapeDtypeStruct((N, H * D), jnp.float32),
    )(acc2)


def kernel(x, edge_attr, edge_index, WQ, bQ, WK, bK, WV, bV, WE, bE):
    Qh, KVh = _node_proj(x, WQ, bQ, WK, bK, WV, bV)
    Eh = _edge_proj(edge_attr, WE, bE)
    src = edge_index[0]
    dst = edge_index[1]
    pad = ((0, 0), (0, NCHUNK_PAD - NCHUNK), (0, 0))
    src3 = jnp.pad(src.reshape(NW, NCHUNK, CH), pad).reshape(NW * NCHUNK_PAD, CH)
    dst3 = jnp.pad(dst.reshape(NW, NCHUNK, CH), pad).reshape(NW * NCHUNK_PAD, CH)
    oacc, = _edge_stage4(src3, dst3, Qh, KVh, Eh)
    out = _finalize(oacc)
    return out.reshape(N, H, D)
